# baseline (speedup 1.0000x reference)
import jax
import jax.numpy as jnp
from jax import lax
from jax.experimental import pallas as pl
from jax.experimental.pallas import tpu as pltpu

_N = 2048
_V = 32000
_C = 8
_F = 4
_SLOTS = _C * _F
_LANES = 128


def _body(tiles_ref, cs_ref, fs_ref, lanes_ref,
          logits_ref, w_ref, m2c_ref, fl_ref, mlm_ref, lab_ref,
          loss_ref, pred_ref, acc_ref):
    s = pl.program_id(0)

    @pl.when(s == 0)
    def _init():
        acc_ref[...] = jnp.zeros((_C, _N), jnp.float32)

    c = cs_ref[s]
    f = fs_ref[s]
    ln = lanes_ref[s]
    keep = (m2c_ref[c, f] > 0).astype(jnp.float32)
    wk = w_ref[c, f] * keep

    row_i = lax.broadcasted_iota(jnp.int32, (_C, _LANES), 0)
    lane_i = lax.broadcasted_iota(jnp.int32, (_C, _LANES), 1)
    onehot = jnp.where((row_i == c) & (lane_i == ln), wk, 0.0)

    acc_ref[...] += lax.dot_general(
        onehot, logits_ref[...],
        (((1,), (1,)), ((), ())),
        precision=lax.Precision.HIGHEST,
        preferred_element_type=jnp.float32,
    )

    @pl.when(s == _SLOTS - 1)
    def _finish():
        mask = mlm_ref[...] >= 0
        fl = fl_ref[...]
        scores = jnp.where(mask, acc_ref[...] / fl, 0.0)

        best = scores[0:1, :]
        pred = jnp.zeros((1, _N), jnp.int32)
        for cc in range(1, _C):
            row = scores[cc:cc + 1, :]
            upd = row > best
            best = jnp.where(upd, row, best)
            pred = jnp.where(upd, cc, pred)

        se = jnp.zeros((1, _N), jnp.float32)
        for cc in range(_C):
            se = se + jnp.exp(scores[cc:cc + 1, :] - best)
        lse = jnp.log(se) + best

        lab = lab_ref[...]
        s_lab = jnp.zeros((1, _N), jnp.float32)
        for cc in range(_C):
            s_lab = s_lab + jnp.where(lab == cc, scores[cc:cc + 1, :], 0.0)

        loss_ref[0, 0] = jnp.sum(lse - s_lab) / float(_N)
        pred_ref[...] = pred


def kernel(logits, mlm_labels, labels, weight, m2c, filler_len):
    logits2d = logits.reshape(_N, _V)
    fidx = jnp.maximum(m2c.reshape(-1), 0).astype(jnp.int32)
    tile = fidx // _LANES
    lane = fidx % _LANES
    order = jnp.argsort(tile).astype(jnp.int32)
    tiles_sorted = tile[order]
    lanes_sorted = lane[order]
    cs = order // _F
    fs = order % _F

    grid_spec = pltpu.PrefetchScalarGridSpec(
        num_scalar_prefetch=4,
        grid=(_SLOTS,),
        in_specs=[
            pl.BlockSpec((_N, _LANES), lambda s, T, C, F, L: (0, T[s])),
            pl.BlockSpec(memory_space=pltpu.SMEM),
            pl.BlockSpec(memory_space=pltpu.SMEM),
            pl.BlockSpec(memory_space=pltpu.VMEM),
            pl.BlockSpec(memory_space=pltpu.VMEM),
            pl.BlockSpec(memory_space=pltpu.VMEM),
        ],
        out_specs=[
            pl.BlockSpec(memory_space=pltpu.SMEM),
            pl.BlockSpec(memory_space=pltpu.VMEM),
        ],
        scratch_shapes=[pltpu.VMEM((_C, _N), jnp.float32)],
    )

    loss, pred = pl.pallas_call(
        _body,
        grid_spec=grid_spec,
        out_shape=[
            jax.ShapeDtypeStruct((1, 1), jnp.float32),
            jax.ShapeDtypeStruct((1, _N), jnp.int32),
        ],
    )(
        tiles_sorted, cs, fs, lanes_sorted,
        logits2d,
        weight,
        m2c,
        filler_len.reshape(_C, 1),
        mlm_labels.reshape(1, _N),
        labels.reshape(1, _N).astype(jnp.int32),
    )
    return loss[0, 0], pred.reshape(_N)
